# baseline (speedup 1.0000x reference)
import jax
import jax.numpy as jnp
from jax import lax
from jax.experimental import pallas as pl
from jax.experimental.pallas import tpu as pltpu

NUM_FRAMES = 4096
NUM_CLASSES = 1024
CH = 256
NCH = NUM_FRAMES // CH
NBUF = 6
AHEAD = 5
BLANK = 0
NEG = -2147483648


def _chunk_argmax(x):
    m = jnp.max(x, axis=1, keepdims=True)
    cls = lax.broadcasted_iota(jnp.int32, x.shape, 1).astype(jnp.float32)
    cand = jnp.where(x == m, cls, jnp.float32(NUM_CLASSES))
    return jnp.min(cand, axis=1).astype(jnp.int32).reshape(1, CH)


def _decode(x_ref, o_ref, b0, b1, b2, b3, b4, b5, s0, s1, s2, s3, s4, s5):
    bufs = (b0, b1, b2, b3, b4, b5)
    sems = (s0, s1, s2, s3, s4, s5)

    def start(k):
        return pltpu.make_async_copy(
            x_ref.at[pl.ds(k * CH, CH), :], bufs[k % NBUF], sems[k % NBUF])

    cps = {}
    for k in range(AHEAD):
        cps[k] = start(k)
        cps[k].start()

    carry = jnp.int32(-1)
    pos = lax.broadcasted_iota(jnp.int32, (1, CH), 1)
    for k in range(NCH):
        if k + AHEAD < NCH:
            cps[k + AHEAD] = start(k + AHEAD)
            cps[k + AHEAD].start()
        cps[k].wait()
        idx = _chunk_argmax(bufs[k % NBUF][...])
        prev = jnp.where(pos == 0, carry, jnp.roll(idx, 1, axis=1))
        keep = (idx != prev) & (idx != BLANK)
        o_ref[pl.ds(0, 1), pl.ds(k * CH, CH)] = jnp.where(
            keep, idx, jnp.int32(-1))
        carry = jnp.max(jnp.where(pos == CH - 1, idx, NEG))


def kernel(emission):
    out = pl.pallas_call(
        _decode,
        in_specs=[pl.BlockSpec(memory_space=pl.ANY)],
        out_specs=pl.BlockSpec(memory_space=pltpu.VMEM),
        out_shape=jax.ShapeDtypeStruct((1, NUM_FRAMES), jnp.int32),
        scratch_shapes=(
            [pltpu.VMEM((CH, NUM_CLASSES), jnp.float32)] * NBUF
            + [pltpu.SemaphoreType.DMA] * NBUF),
    )(emission)
    return out.reshape(NUM_FRAMES)


# speedup vs baseline: 1.2981x; 1.2981x over previous
import jax
import jax.numpy as jnp
from jax import lax
from jax.experimental import pallas as pl
from jax.experimental.pallas import tpu as pltpu

NUM_FRAMES = 4096
NUM_CLASSES = 1024
CH = 512
NCH = NUM_FRAMES // CH
NBUF = 4
AHEAD = 3
BLANK = 0
NEG = -2147483648


def _chunk_argmax(x):
    m = jnp.max(x, axis=1, keepdims=True)
    cls = lax.broadcasted_iota(jnp.int32, x.shape, 1).astype(jnp.float32)
    cand = jnp.where(x == m, cls, jnp.float32(NUM_CLASSES))
    return jnp.min(cand, axis=1).astype(jnp.int32).reshape(1, CH)


def _decode(x_ref, o_ref, b0, b1, b2, b3, s0, s1, s2, s3):
    bufs = (b0, b1, b2, b3)
    sems = (s0, s1, s2, s3)

    def start(k):
        return pltpu.make_async_copy(
            x_ref.at[pl.ds(k * CH, CH), :], bufs[k % NBUF], sems[k % NBUF])

    cps = {}
    for k in range(AHEAD):
        cps[k] = start(k)
        cps[k].start()

    carry = jnp.int32(-1)
    pos = lax.broadcasted_iota(jnp.int32, (1, CH), 1)
    for k in range(NCH):
        if k + AHEAD < NCH:
            cps[k + AHEAD] = start(k + AHEAD)
            cps[k + AHEAD].start()
        cps[k].wait()
        idx = _chunk_argmax(bufs[k % NBUF][...])
        prev = jnp.where(pos == 0, carry, jnp.roll(idx, 1, axis=1))
        keep = (idx != prev) & (idx != BLANK)
        o_ref[pl.ds(0, 1), pl.ds(k * CH, CH)] = jnp.where(
            keep, idx, jnp.int32(-1))
        carry = jnp.max(jnp.where(pos == CH - 1, idx, NEG))


def kernel(emission):
    out = pl.pallas_call(
        _decode,
        in_specs=[pl.BlockSpec(memory_space=pl.ANY)],
        out_specs=pl.BlockSpec(memory_space=pltpu.VMEM),
        out_shape=jax.ShapeDtypeStruct((1, NUM_FRAMES), jnp.int32),
        scratch_shapes=[
            pltpu.VMEM((CH, NUM_CLASSES), jnp.float32),
            pltpu.VMEM((CH, NUM_CLASSES), jnp.float32),
            pltpu.VMEM((CH, NUM_CLASSES), jnp.float32),
            pltpu.VMEM((CH, NUM_CLASSES), jnp.float32),
            pltpu.SemaphoreType.DMA,
            pltpu.SemaphoreType.DMA,
            pltpu.SemaphoreType.DMA,
            pltpu.SemaphoreType.DMA,
        ],
    )(emission)
    return out.reshape(NUM_FRAMES)
